# baseline (speedup 1.0000x reference)
import functools

import jax
import jax.numpy as jnp
from jax import lax
from jax.experimental import pallas as pl
from jax.experimental.pallas import tpu as pltpu
from jax.experimental.pallas import tpu_sc as plsc

_MAX_POSITION = 15000
_NUM_CORES = 2
_NUM_SUBCORES = 16
_NUM_WORKERS = _NUM_CORES * _NUM_SUBCORES
_CHUNK = 128
_LANES = 16
_NSLOT = 8
_LOOKAHEAD = 4
_BLK = _NSLOT * _CHUNK


def kernel(position_ids, pe):
    batch, hist = position_ids.shape
    vocab, dim = pe.shape
    total = batch * hist
    per_worker = total // _NUM_WORKERS
    assert per_worker * _NUM_WORKERS == total
    nchunks = per_worker // _CHUNK
    ngroups = nchunks // _NSLOT
    assert ngroups * _NSLOT == nchunks and ngroups % 2 == 0 and ngroups >= 4

    ids_flat = position_ids.reshape(total)
    mesh = plsc.VectorSubcoreMesh(core_axis_name="c", subcore_axis_name="s")

    @functools.partial(
        pl.kernel,
        mesh=mesh,
        out_type=jax.ShapeDtypeStruct((total, dim), jnp.float32),
        compiler_params=pltpu.CompilerParams(use_tc_tiling_on_sc=False),
        scratch_types=[
            pltpu.VMEM((2, _BLK), jnp.int32),
            pltpu.VMEM((_NSLOT, _CHUNK, dim), jnp.float32),
            pltpu.VMEM_SHARED((vocab, dim), jnp.float32),
            pltpu.SemaphoreType.DMA((2,)),
            pltpu.SemaphoreType.DMA((_NSLOT,)),
            pltpu.SemaphoreType.DMA((_NSLOT,)),
        ],
    )
    def run(ids_hbm, pe_hbm, out_hbm, iblk, rows, pe_sh, isem, gsem, psem):
        sid = lax.axis_index("s")
        wid = sid * _NUM_CORES + lax.axis_index("c")
        base = pl.multiple_of(wid * per_worker, _CHUNK)

        @pl.when(sid == 0)
        def _stage():
            pltpu.sync_copy(pe_hbm, pe_sh)

        plsc.subcore_barrier()

        def load_block(u, slot):
            src = ids_hbm.at[pl.ds(base + u * _BLK, _BLK)]
            pltpu.make_async_copy(src, iblk.at[slot], isem.at[slot]).start()

        def wait_block(slot):
            src = ids_hbm.at[pl.ds(base, _BLK)]
            pltpu.make_async_copy(src, iblk.at[slot], isem.at[slot]).wait()

        def clamp(bslot, pos):
            for j in range(_CHUNK // _LANES):
                sl = pl.ds(pos * _CHUNK + j * _LANES, _LANES)
                iblk[bslot, sl] = jnp.minimum(iblk[bslot, sl], _MAX_POSITION - 1)

        def start_gather(c, slot, bslot, pos):
            idx_sl = iblk.at[bslot, pl.ds(pos * _CHUNK, _CHUNK)]
            src = pe_sh if slot % 2 == 0 else pe_hbm
            pltpu.make_async_copy(src.at[idx_sl], rows.at[slot], gsem.at[slot]).start()

        def wait_gather(slot):
            src = pe_hbm.at[pl.ds(0, _CHUNK)]
            pltpu.make_async_copy(src, rows.at[slot], gsem.at[slot]).wait()

        def start_put(c, slot):
            dst = out_hbm.at[pl.ds(base + c * _CHUNK, _CHUNK)]
            pltpu.make_async_copy(rows.at[slot], dst, psem.at[slot]).start()

        def wait_put(slot):
            dst = out_hbm.at[pl.ds(base, _CHUNK)]
            pltpu.make_async_copy(rows.at[slot], dst, psem.at[slot]).wait()

        def body(c, b, cur, do_wait_put=True, prefetch=True):
            wait_gather(b)
            start_put(c, b)
            if prefetch:
                ps = (b + _LOOKAHEAD) % _NSLOT
                bs = cur if b < _LOOKAHEAD else (1 - cur)
                pos = (b + _LOOKAHEAD) % _NSLOT
                if b == _LOOKAHEAD:
                    wait_block(1 - cur)
                clamp(bs, pos)
                if do_wait_put:
                    wait_put(ps)
                start_gather(c + _LOOKAHEAD, ps, bs, pos)

        load_block(0, 0)
        wait_block(0)
        for b in range(_LOOKAHEAD):
            clamp(0, b)
            start_gather(b, b, 0, b)

        load_block(1, 1)
        for b in range(_NSLOT):
            body(b, b, cur=0, do_wait_put=(b >= _LOOKAHEAD))

        def pair(i, carry):
            for k in range(2):
                u = 2 * i + 1 + k
                cur = (1 + k) % 2
                load_block(u + 1, 1 - cur)
                c0 = u * _NSLOT
                for b in range(_NSLOT):
                    body(c0 + b, b, cur=cur)
            return carry

        lax.fori_loop(0, (ngroups - 2) // 2, pair, None)

        c0 = (ngroups - 1) * _NSLOT
        last_cur = (ngroups - 1) % 2
        for b in range(_NSLOT):
            body(c0 + b, b, cur=last_cur, prefetch=(b < _LOOKAHEAD))

        for b in range(_NSLOT):
            wait_put(b)

    out = run(ids_flat, pe)
    return out.reshape(batch, hist, dim)


# speedup vs baseline: 1.0666x; 1.0666x over previous
import functools

import jax
import jax.numpy as jnp
from jax import lax
from jax.experimental import pallas as pl
from jax.experimental.pallas import tpu as pltpu
from jax.experimental.pallas import tpu_sc as plsc

_MAX_POSITION = 15000
_NUM_CORES = 2
_NUM_SUBCORES = 16
_NUM_WORKERS = _NUM_CORES * _NUM_SUBCORES
_CHUNK = 512
_LANES = 16
_CPB = 4
_BLK = _CPB * _CHUNK


def kernel(position_ids, pe):
    batch, hist = position_ids.shape
    vocab, dim = pe.shape
    total = batch * hist
    per_worker = total // _NUM_WORKERS
    assert per_worker * _NUM_WORKERS == total
    nchunks = per_worker // _CHUNK
    nblocks = nchunks // _CPB
    assert nblocks * _CPB == nchunks and nblocks % 2 == 0 and nblocks >= 4

    ids_flat = position_ids.reshape(total)
    mesh = plsc.VectorSubcoreMesh(core_axis_name="c", subcore_axis_name="s")

    @functools.partial(
        pl.kernel,
        mesh=mesh,
        out_type=jax.ShapeDtypeStruct((total, dim), jnp.float32),
        compiler_params=pltpu.CompilerParams(use_tc_tiling_on_sc=False),
        scratch_types=[
            pltpu.VMEM((2, _BLK), jnp.int32),
            pltpu.VMEM((2, _CHUNK, dim), jnp.float32),
            pltpu.VMEM_SHARED((vocab, dim), jnp.float32),
            pltpu.SemaphoreType.DMA((2,)),
            pltpu.SemaphoreType.DMA((2,)),
            pltpu.SemaphoreType.DMA((2,)),
        ],
    )
    def run(ids_hbm, pe_hbm, out_hbm, iblk, rows, pe_sh, isem, gsem, psem):
        sid = lax.axis_index("s")
        wid = sid * _NUM_CORES + lax.axis_index("c")
        base = pl.multiple_of(wid * per_worker, _CHUNK)

        @pl.when(sid == 0)
        def _stage():
            pltpu.sync_copy(pe_hbm, pe_sh)

        plsc.subcore_barrier()

        def load_block(u, slot):
            src = ids_hbm.at[pl.ds(base + u * _BLK, _BLK)]
            pltpu.make_async_copy(src, iblk.at[slot], isem.at[slot]).start()

        def wait_block(slot):
            src = ids_hbm.at[pl.ds(base, _BLK)]
            pltpu.make_async_copy(src, iblk.at[slot], isem.at[slot]).wait()

        def clamp_block(slot):
            for j in range(_BLK // _LANES):
                sl = pl.ds(j * _LANES, _LANES)
                iblk[slot, sl] = jnp.minimum(iblk[slot, sl], _MAX_POSITION - 1)

        def start_gather(c, s, bslot, pos):
            idx_sl = iblk.at[bslot, pl.ds(pos * _CHUNK, _CHUNK)]
            pltpu.make_async_copy(pe_sh.at[idx_sl], rows.at[s], gsem.at[s]).start()

        def wait_gather(s):
            src = pe_hbm.at[pl.ds(0, _CHUNK)]
            pltpu.make_async_copy(src, rows.at[s], gsem.at[s]).wait()

        def start_put(c, s):
            dst = out_hbm.at[pl.ds(base + c * _CHUNK, _CHUNK)]
            pltpu.make_async_copy(rows.at[s], dst, psem.at[s]).start()

        def wait_put(s):
            dst = out_hbm.at[pl.ds(base, _CHUNK)]
            pltpu.make_async_copy(rows.at[s], dst, psem.at[s]).wait()

        def step(c, b, k):
            s = b % 2
            wait_put(s)
            start_gather(c, s, k, b)
            wait_gather(1 - s)
            start_put(c - 1, 1 - s)

        load_block(0, 0)
        load_block(1, 1)
        wait_block(0)
        clamp_block(0)
        start_gather(0, 0, 0, 0)
        start_gather(1, 1, 0, 1)
        wait_gather(0)
        start_put(0, 0)
        step(2, 2, 0)
        step(3, 3, 0)

        def macro(m, k):
            wait_block(k)
            clamp_block(k)
            c0 = m * _CPB
            step(c0, 0, k)
            load_block(m + 1, 1 - k)
            step(c0 + 1, 1, k)
            step(c0 + 2, 2, k)
            step(c0 + 3, 3, k)

        def pair(i, carry):
            macro(2 * i + 1, 1)
            macro(2 * i + 2, 0)
            return carry

        lax.fori_loop(0, (nblocks - 2) // 2, pair, None)

        m = nblocks - 1
        k = m % 2
        wait_block(k)
        clamp_block(k)
        c0 = m * _CPB
        for b in range(_CPB):
            step(c0 + b, b, k)

        wait_gather(1)
        start_put(nchunks - 1, 1)
        wait_put(0)
        wait_put(1)

    out = run(ids_flat, pe)
    return out.reshape(batch, hist, dim)


# speedup vs baseline: 1.0721x; 1.0052x over previous
import functools

import jax
import jax.numpy as jnp
from jax import lax
from jax.experimental import pallas as pl
from jax.experimental.pallas import tpu as pltpu
from jax.experimental.pallas import tpu_sc as plsc

_MAX_POSITION = 15000
_NUM_CORES = 2
_NUM_SUBCORES = 16
_NUM_WORKERS = _NUM_CORES * _NUM_SUBCORES
_CHUNK = 128
_LANES = 16
_NSLOT = 8
_LOOKAHEAD = 4
_BLK = _NSLOT * _CHUNK


def kernel(position_ids, pe):
    batch, hist = position_ids.shape
    vocab, dim = pe.shape
    total = batch * hist
    per_worker = total // _NUM_WORKERS
    assert per_worker * _NUM_WORKERS == total
    nchunks = per_worker // _CHUNK
    ngroups = nchunks // _NSLOT
    assert ngroups * _NSLOT == nchunks and ngroups % 2 == 0 and ngroups >= 4

    ids_flat = position_ids.reshape(total)
    mesh = plsc.VectorSubcoreMesh(core_axis_name="c", subcore_axis_name="s")

    @functools.partial(
        pl.kernel,
        mesh=mesh,
        out_type=jax.ShapeDtypeStruct((total, dim), jnp.float32),
        compiler_params=pltpu.CompilerParams(use_tc_tiling_on_sc=False),
        scratch_types=[
            pltpu.VMEM((2, _BLK), jnp.int32),
            pltpu.VMEM((_NSLOT, _CHUNK, dim), jnp.float32),
            pltpu.VMEM_SHARED((vocab, dim), jnp.float32),
            pltpu.SemaphoreType.DMA((2,)),
            pltpu.SemaphoreType.DMA((_NSLOT,)),
            pltpu.SemaphoreType.DMA((_NSLOT,)),
        ],
    )
    def run(ids_hbm, pe_hbm, out_hbm, iblk, rows, pe_sh, isem, gsem, psem):
        sid = lax.axis_index("s")
        wid = sid * _NUM_CORES + lax.axis_index("c")
        base = pl.multiple_of(wid * per_worker, _CHUNK)

        @pl.when(sid == 0)
        def _stage():
            pltpu.sync_copy(pe_hbm, pe_sh)

        plsc.subcore_barrier()

        def load_block(u, slot):
            src = ids_hbm.at[pl.ds(base + u * _BLK, _BLK)]
            pltpu.make_async_copy(src, iblk.at[slot], isem.at[slot]).start()

        def wait_block(slot):
            src = ids_hbm.at[pl.ds(base, _BLK)]
            pltpu.make_async_copy(src, iblk.at[slot], isem.at[slot]).wait()

        def clamp(bslot, pos):
            for j in range(_CHUNK // _LANES):
                sl = pl.ds(pos * _CHUNK + j * _LANES, _LANES)
                iblk[bslot, sl] = jnp.minimum(iblk[bslot, sl], _MAX_POSITION - 1)

        def start_gather(c, slot, bslot, pos):
            idx_sl = iblk.at[bslot, pl.ds(pos * _CHUNK, _CHUNK)]
            pltpu.make_async_copy(pe_sh.at[idx_sl], rows.at[slot], gsem.at[slot]).start()

        def wait_gather(slot):
            src = pe_hbm.at[pl.ds(0, _CHUNK)]
            pltpu.make_async_copy(src, rows.at[slot], gsem.at[slot]).wait()

        def start_put(c, slot):
            dst = out_hbm.at[pl.ds(base + c * _CHUNK, _CHUNK)]
            pltpu.make_async_copy(rows.at[slot], dst, psem.at[slot]).start()

        def wait_put(slot):
            dst = out_hbm.at[pl.ds(base, _CHUNK)]
            pltpu.make_async_copy(rows.at[slot], dst, psem.at[slot]).wait()

        def body(c, b, cur, do_wait_put=True, prefetch=True):
            wait_gather(b)
            start_put(c, b)
            if prefetch:
                ps = (b + _LOOKAHEAD) % _NSLOT
                bs = cur if b < _LOOKAHEAD else (1 - cur)
                pos = (b + _LOOKAHEAD) % _NSLOT
                if b == _LOOKAHEAD:
                    wait_block(1 - cur)
                clamp(bs, pos)
                if do_wait_put:
                    wait_put(ps)
                start_gather(c + _LOOKAHEAD, ps, bs, pos)

        load_block(0, 0)
        wait_block(0)
        for b in range(_LOOKAHEAD):
            clamp(0, b)
            start_gather(b, b, 0, b)

        load_block(1, 1)
        for b in range(_NSLOT):
            body(b, b, cur=0, do_wait_put=(b >= _LOOKAHEAD))

        def pair(i, carry):
            for k in range(2):
                u = 2 * i + 1 + k
                cur = (1 + k) % 2
                load_block(u + 1, 1 - cur)
                c0 = u * _NSLOT
                for b in range(_NSLOT):
                    body(c0 + b, b, cur=cur)
            return carry

        lax.fori_loop(0, (ngroups - 2) // 2, pair, None)

        c0 = (ngroups - 1) * _NSLOT
        last_cur = (ngroups - 1) % 2
        for b in range(_NSLOT):
            body(c0 + b, b, cur=last_cur, prefetch=(b < _LOOKAHEAD))

        for b in range(_NSLOT):
            wait_put(b)

    out = run(ids_flat, pe)
    return out.reshape(batch, hist, dim)


# speedup vs baseline: 1.0723x; 1.0002x over previous
import functools

import jax
import jax.numpy as jnp
from jax import lax
from jax.experimental import pallas as pl
from jax.experimental.pallas import tpu as pltpu
from jax.experimental.pallas import tpu_sc as plsc

_MAX_POSITION = 15000
_NUM_CORES = 2
_NUM_SUBCORES = 16
_NUM_WORKERS = _NUM_CORES * _NUM_SUBCORES
_CHUNK = 128
_LANES = 16
_NSLOT = 8
_LOOKAHEAD = 6
_BLK = _NSLOT * _CHUNK


def kernel(position_ids, pe):
    batch, hist = position_ids.shape
    vocab, dim = pe.shape
    total = batch * hist
    per_worker = total // _NUM_WORKERS
    assert per_worker * _NUM_WORKERS == total
    nchunks = per_worker // _CHUNK
    ngroups = nchunks // _NSLOT
    assert ngroups * _NSLOT == nchunks and ngroups % 2 == 0 and ngroups >= 4

    ids_flat = position_ids.reshape(total)
    mesh = plsc.VectorSubcoreMesh(core_axis_name="c", subcore_axis_name="s")

    @functools.partial(
        pl.kernel,
        mesh=mesh,
        out_type=jax.ShapeDtypeStruct((total, dim), jnp.float32),
        compiler_params=pltpu.CompilerParams(use_tc_tiling_on_sc=False),
        scratch_types=[
            pltpu.VMEM((2, _BLK), jnp.int32),
            pltpu.VMEM((_NSLOT, _CHUNK, dim), jnp.float32),
            pltpu.VMEM_SHARED((vocab, dim), jnp.float32),
            pltpu.SemaphoreType.DMA((2,)),
            pltpu.SemaphoreType.DMA((_NSLOT,)),
            pltpu.SemaphoreType.DMA((_NSLOT,)),
        ],
    )
    def run(ids_hbm, pe_hbm, out_hbm, iblk, rows, pe_sh, isem, gsem, psem):
        sid = lax.axis_index("s")
        wid = sid * _NUM_CORES + lax.axis_index("c")
        base = pl.multiple_of(wid * per_worker, _CHUNK)

        @pl.when(sid == 0)
        def _stage():
            pltpu.sync_copy(pe_hbm, pe_sh)

        plsc.subcore_barrier()

        def load_block(u, slot):
            src = ids_hbm.at[pl.ds(base + u * _BLK, _BLK)]
            pltpu.make_async_copy(src, iblk.at[slot], isem.at[slot]).start()

        def wait_block(slot):
            src = ids_hbm.at[pl.ds(base, _BLK)]
            pltpu.make_async_copy(src, iblk.at[slot], isem.at[slot]).wait()

        def clamp(bslot, pos):
            for j in range(_CHUNK // _LANES):
                sl = pl.ds(pos * _CHUNK + j * _LANES, _LANES)
                iblk[bslot, sl] = jnp.minimum(iblk[bslot, sl], _MAX_POSITION - 1)

        def start_gather(c, slot, bslot, pos):
            idx_sl = iblk.at[bslot, pl.ds(pos * _CHUNK, _CHUNK)]
            pltpu.make_async_copy(pe_sh.at[idx_sl], rows.at[slot], gsem.at[slot]).start()

        def wait_gather(slot):
            src = pe_hbm.at[pl.ds(0, _CHUNK)]
            pltpu.make_async_copy(src, rows.at[slot], gsem.at[slot]).wait()

        def start_put(c, slot):
            dst = out_hbm.at[pl.ds(base + c * _CHUNK, _CHUNK)]
            pltpu.make_async_copy(rows.at[slot], dst, psem.at[slot]).start()

        def wait_put(slot):
            dst = out_hbm.at[pl.ds(base, _CHUNK)]
            pltpu.make_async_copy(rows.at[slot], dst, psem.at[slot]).wait()

        def body(c, b, cur, do_wait_put=True, prefetch=True):
            wait_gather(b)
            start_put(c, b)
            if prefetch:
                ps = (b + _LOOKAHEAD) % _NSLOT
                bs = cur if b < _NSLOT - _LOOKAHEAD else (1 - cur)
                pos = (b + _LOOKAHEAD) % _NSLOT
                if b == _NSLOT - _LOOKAHEAD:
                    wait_block(1 - cur)
                clamp(bs, pos)
                if do_wait_put:
                    wait_put(ps)
                start_gather(c + _LOOKAHEAD, ps, bs, pos)

        load_block(0, 0)
        wait_block(0)
        for b in range(_LOOKAHEAD):
            clamp(0, b)
            start_gather(b, b, 0, b)

        load_block(1, 1)
        for b in range(_NSLOT):
            body(b, b, cur=0, do_wait_put=(b >= _NSLOT - _LOOKAHEAD))

        def pair(i, carry):
            for k in range(2):
                u = 2 * i + 1 + k
                cur = (1 + k) % 2
                load_block(u + 1, 1 - cur)
                c0 = u * _NSLOT
                for b in range(_NSLOT):
                    body(c0 + b, b, cur=cur)
            return carry

        lax.fori_loop(0, (ngroups - 2) // 2, pair, None)

        c0 = (ngroups - 1) * _NSLOT
        last_cur = (ngroups - 1) % 2
        for b in range(_NSLOT):
            body(c0 + b, b, cur=last_cur, prefetch=(b < _NSLOT - _LOOKAHEAD))

        for b in range(_NSLOT):
            wait_put(b)

    out = run(ids_flat, pe)
    return out.reshape(batch, hist, dim)
